# baseline (speedup 1.0000x reference)
import jax
import jax.numpy as jnp
from jax.experimental import pallas as pl

_F = 6600
_J = 22
_TN = 40
_G = 8
_W = _G * _TN * _J
_A = _G * _J
_R_BLK = 165
_NR = _F // (_G * _R_BLK)


def _fwd_kernel(fr_ref, flx_ref, fly_ref, tpat_ref, e_ref,
                sig_ref, amax_ref, smax_ref, reax_ref, out_ref):
    fr = fr_ref[0]
    x = fr[:, 1:2]
    y = fr[:, 2:3]
    vx = fr[:, 3:4]
    vy = fr[:, 4:5]
    ax = fr[:, 5:6]
    ay = fr[:, 6:7]

    sigma = sig_ref[0, 0]
    a_max = amax_ref[0, 0]
    s_max = smax_ref[0, 0]
    reax_t = reax_ref[0, 0]

    vxr = ax * reax_t + vx
    vyr = ay * reax_t + vy
    xr = x + vx * reax_t + 0.5 * ax * reax_t * reax_t
    yr = y + vy * reax_t + 0.5 * ay * reax_t * reax_t

    flx = flx_ref[0]
    fly = fly_ref[0]

    dx = flx - xr
    dy = fly - yr
    dmag = jnp.sqrt(dx * dx + dy * dy)
    s0 = jnp.clip((dx * vxr + dy * vyr) / dmag, -s_max, s_max)
    t_lt = (s_max - s0) / a_max
    d_lt = t_lt * (s0 + s_max) * 0.5
    soa = s0 / a_max
    t_lt = jnp.where(d_lt > dmag,
                     -soa + jnp.sqrt(soa * soa + 2.0 * dmag / a_max),
                     t_lt)
    d_lt = jnp.maximum(jnp.minimum(d_lt, dmag), 0.0)
    t_tot = reax_t + t_lt + (dmag - d_lt) / s_max

    kk = (jnp.pi / jnp.sqrt(3.0)) / sigma
    z = kk * t_tot

    z_hi = z.astype(jnp.bfloat16)
    z_lo = (z - z_hi.astype(jnp.float32)).astype(jnp.bfloat16)
    e = e_ref[...]
    dn = (((0,), (0,)), ((), ()))
    zexp = (jax.lax.dot_general(z_hi, e, dn,
                                preferred_element_type=jnp.float32)
            + jax.lax.dot_general(z_lo, e, dn,
                                  preferred_element_type=jnp.float32))

    u = kk * tpat_ref[...] - zexp
    out_ref[0, 0] = jax.nn.sigmoid(u)


def kernel(frame, tti_sigma, a_max, s_max, reax_t):
    B = frame.shape[0]

    x = jnp.linspace(0.5, 119.5, 120).astype(jnp.float32)
    y = jnp.linspace(-0.5, 53.5, 55).astype(jnp.float32)
    y = y.at[0].set(-0.2)
    yy, xx = jnp.meshgrid(y, x, indexing='ij')
    flx_flat = xx.reshape(_F)
    fly_flat = yy.reshape(_F)

    a_idx = jnp.arange(_A)
    r_idx = jnp.arange(_NR * _R_BLK)
    fidx = _G * r_idx[None, :] + (a_idx // _J)[:, None]
    flx_pat = flx_flat[fidx].reshape(_A, _NR, _R_BLK).transpose(1, 0, 2)
    fly_pat = fly_flat[fidx].reshape(_A, _NR, _R_BLK).transpose(1, 0, 2)

    T = jnp.linspace(0.1, 4.0, _TN).astype(jnp.float32)
    q = jnp.arange(_W)
    tpat = T[(q % (_TN * _J)) // _J].reshape(1, _W)
    e = ((q[None, :] // (_TN * _J) == (a_idx // _J)[:, None])
         & (q[None, :] % _J == (a_idx % _J)[:, None])).astype(jnp.bfloat16)

    frame_t = jnp.tile(frame, (1, _G, 1))

    def s11(v):
        return jnp.asarray(v, jnp.float32).reshape(1, 1)

    out = pl.pallas_call(
        _fwd_kernel,
        grid=(B, _NR),
        in_specs=[
            pl.BlockSpec((1, _A, 12), lambda b, f: (b, 0, 0)),
            pl.BlockSpec((1, _A, _R_BLK), lambda b, f: (f, 0, 0)),
            pl.BlockSpec((1, _A, _R_BLK), lambda b, f: (f, 0, 0)),
            pl.BlockSpec((1, _W), lambda b, f: (0, 0)),
            pl.BlockSpec((_A, _W), lambda b, f: (0, 0)),
            pl.BlockSpec((1, 1), lambda b, f: (0, 0)),
            pl.BlockSpec((1, 1), lambda b, f: (0, 0)),
            pl.BlockSpec((1, 1), lambda b, f: (0, 0)),
            pl.BlockSpec((1, 1), lambda b, f: (0, 0)),
        ],
        out_specs=pl.BlockSpec((1, 1, _R_BLK, _W), lambda b, f: (b, f, 0, 0)),
        out_shape=jax.ShapeDtypeStruct((B, _NR, _R_BLK, _W), jnp.float32),
    )(frame_t, flx_pat, fly_pat, tpat, e,
      s11(tti_sigma), s11(a_max), s11(s_max), s11(reax_t))

    return out.reshape(B, _F, _TN, _J)


# speedup vs baseline: 2.1515x; 2.1515x over previous
import jax
import jax.numpy as jnp
import numpy as np
from jax.experimental import pallas as pl

_F = 6600
_J = 22
_TN = 40
_G = 8
_W = _G * _TN * _J
_A = _G * _J
_R_BLK = 165
_NR = _F // (_G * _R_BLK)


def _fwd_kernel(fr_ref, flx_ref, fly_ref, tpat_ref, e_ref,
                sig_ref, amax_ref, smax_ref, reax_ref, out_ref):
    fr = fr_ref[0]
    x = fr[:, 1:2]
    y = fr[:, 2:3]
    vx = fr[:, 3:4]
    vy = fr[:, 4:5]
    ax = fr[:, 5:6]
    ay = fr[:, 6:7]

    sigma = sig_ref[0, 0]
    a_max = amax_ref[0, 0]
    s_max = smax_ref[0, 0]
    reax_t = reax_ref[0, 0]

    vxr = ax * reax_t + vx
    vyr = ay * reax_t + vy
    xr = x + vx * reax_t + 0.5 * ax * reax_t * reax_t
    yr = y + vy * reax_t + 0.5 * ay * reax_t * reax_t

    flx = flx_ref[0]
    fly = fly_ref[0]

    dx = flx - xr
    dy = fly - yr
    dmag = jnp.sqrt(dx * dx + dy * dy)
    s0 = jnp.clip((dx * vxr + dy * vyr) / dmag, -s_max, s_max)
    t_lt = (s_max - s0) / a_max
    d_lt = t_lt * (s0 + s_max) * 0.5
    soa = s0 / a_max
    t_lt = jnp.where(d_lt > dmag,
                     -soa + jnp.sqrt(soa * soa + 2.0 * dmag / a_max),
                     t_lt)
    d_lt = jnp.maximum(jnp.minimum(d_lt, dmag), 0.0)
    t_tot = reax_t + t_lt + (dmag - d_lt) / s_max

    kk = (jnp.pi / jnp.sqrt(3.0)) / sigma
    z = kk * t_tot

    z_hi = z.astype(jnp.bfloat16)
    z_lo = (z - z_hi.astype(jnp.float32)).astype(jnp.bfloat16)
    e = e_ref[...]
    dn = (((0,), (0,)), ((), ()))
    zexp = (jax.lax.dot_general(z_hi, e, dn,
                                preferred_element_type=jnp.float32)
            + jax.lax.dot_general(z_lo, e, dn,
                                  preferred_element_type=jnp.float32))

    u = kk * tpat_ref[...] - zexp
    out_ref[0, 0] = jax.nn.sigmoid(u)


def kernel(frame, tti_sigma, a_max, s_max, reax_t):
    B = frame.shape[0]

    x = np.linspace(0.5, 119.5, 120, dtype=np.float32)
    y = np.linspace(-0.5, 53.5, 55, dtype=np.float32)
    y[0] = -0.2
    yy, xx = np.meshgrid(y, x, indexing='ij')
    flx_flat = xx.reshape(_F).astype(np.float32)
    fly_flat = yy.reshape(_F).astype(np.float32)

    a_idx = np.arange(_A)
    r_idx = np.arange(_NR * _R_BLK)
    fidx = _G * r_idx[None, :] + (a_idx // _J)[:, None]
    flx_pat = jnp.asarray(
        flx_flat[fidx].reshape(_A, _NR, _R_BLK).transpose(1, 0, 2))
    fly_pat = jnp.asarray(
        fly_flat[fidx].reshape(_A, _NR, _R_BLK).transpose(1, 0, 2))

    T = np.linspace(0.1, 4.0, _TN, dtype=np.float32)
    q = np.arange(_W)
    tpat = jnp.asarray(T[(q % (_TN * _J)) // _J].reshape(1, _W))
    e = jnp.asarray(
        ((q[None, :] // (_TN * _J) == (a_idx // _J)[:, None])
         & (q[None, :] % _J == (a_idx % _J)[:, None])), dtype=jnp.bfloat16)

    frame_t = jnp.tile(frame, (1, _G, 1))

    def s11(v):
        return jnp.asarray(v, jnp.float32).reshape(1, 1)

    out = pl.pallas_call(
        _fwd_kernel,
        grid=(B, _NR),
        in_specs=[
            pl.BlockSpec((1, _A, 12), lambda b, f: (b, 0, 0)),
            pl.BlockSpec((1, _A, _R_BLK), lambda b, f: (f, 0, 0)),
            pl.BlockSpec((1, _A, _R_BLK), lambda b, f: (f, 0, 0)),
            pl.BlockSpec((1, _W), lambda b, f: (0, 0)),
            pl.BlockSpec((_A, _W), lambda b, f: (0, 0)),
            pl.BlockSpec((1, 1), lambda b, f: (0, 0)),
            pl.BlockSpec((1, 1), lambda b, f: (0, 0)),
            pl.BlockSpec((1, 1), lambda b, f: (0, 0)),
            pl.BlockSpec((1, 1), lambda b, f: (0, 0)),
        ],
        out_specs=pl.BlockSpec((1, 1, _R_BLK, _W), lambda b, f: (b, f, 0, 0)),
        out_shape=jax.ShapeDtypeStruct((B, _NR, _R_BLK, _W), jnp.float32),
    )(frame_t, flx_pat, fly_pat, tpat, e,
      s11(tti_sigma), s11(a_max), s11(s_max), s11(reax_t))

    return out.reshape(B, _F, _TN, _J)


# speedup vs baseline: 83.8948x; 38.9943x over previous
import jax
import jax.numpy as jnp
import numpy as np
from jax.experimental import pallas as pl

_F = 6600
_J = 22
_TN = 40
_F_BLK = 1664
_NF = 4


def _fwd_kernel(fr_ref, flx_ref, fly_ref, t_ref,
                sig_ref, amax_ref, smax_ref, reax_ref, out_ref):
    fr = fr_ref[0]
    x = fr[:, 1:2]
    y = fr[:, 2:3]
    vx = fr[:, 3:4]
    vy = fr[:, 4:5]
    ax = fr[:, 5:6]
    ay = fr[:, 6:7]

    sigma = sig_ref[0, 0]
    a_max = amax_ref[0, 0]
    s_max = smax_ref[0, 0]
    reax_t = reax_ref[0, 0]

    vxr = ax * reax_t + vx
    vyr = ay * reax_t + vy
    xr = x + vx * reax_t + 0.5 * ax * reax_t * reax_t
    yr = y + vy * reax_t + 0.5 * ay * reax_t * reax_t

    flx = flx_ref[0]
    fly = fly_ref[0]

    dx = flx - xr
    dy = fly - yr
    dmag = jnp.sqrt(dx * dx + dy * dy)
    s0 = jnp.clip((dx * vxr + dy * vyr) / dmag, -s_max, s_max)
    t_lt = (s_max - s0) / a_max
    d_lt = t_lt * (s0 + s_max) * 0.5
    soa = s0 / a_max
    t_lt = jnp.where(d_lt > dmag,
                     -soa + jnp.sqrt(soa * soa + 2.0 * dmag / a_max),
                     t_lt)
    d_lt = jnp.maximum(jnp.minimum(d_lt, dmag), 0.0)
    t_tot = reax_t + t_lt + (dmag - d_lt) / s_max

    kk = (jnp.pi / jnp.sqrt(3.0)) / sigma
    ez = jnp.exp(kk * t_tot)
    ct = jnp.exp(-kk * t_ref[...])

    w = ct.reshape(1, _TN, 1) * ez.reshape(_J, 1, _F_BLK) + 1.0
    out_ref[0] = 1.0 / w


def kernel(frame, tti_sigma, a_max, s_max, reax_t):
    B = frame.shape[0]

    x = np.linspace(0.5, 119.5, 120, dtype=np.float32)
    y = np.linspace(-0.5, 53.5, 55, dtype=np.float32)
    y[0] = -0.2
    yy, xx = np.meshgrid(y, x, indexing='ij')
    pad = _NF * _F_BLK - _F
    flx = jnp.asarray(np.pad(xx.reshape(_F), (0, pad), mode='edge')
                      .reshape(_NF, 1, _F_BLK))
    fly = jnp.asarray(np.pad(yy.reshape(_F), (0, pad), mode='edge')
                      .reshape(_NF, 1, _F_BLK))

    T = np.linspace(0.1, 4.0, _TN, dtype=np.float32)
    tcol = jnp.asarray(T.reshape(_TN, 1))

    def s11(v):
        return jnp.asarray(v, jnp.float32).reshape(1, 1)

    out = pl.pallas_call(
        _fwd_kernel,
        grid=(B, _NF),
        in_specs=[
            pl.BlockSpec((1, _J, 12), lambda b, f: (b, 0, 0)),
            pl.BlockSpec((1, 1, _F_BLK), lambda b, f: (f, 0, 0)),
            pl.BlockSpec((1, 1, _F_BLK), lambda b, f: (f, 0, 0)),
            pl.BlockSpec((_TN, 1), lambda b, f: (0, 0)),
            pl.BlockSpec((1, 1), lambda b, f: (0, 0)),
            pl.BlockSpec((1, 1), lambda b, f: (0, 0)),
            pl.BlockSpec((1, 1), lambda b, f: (0, 0)),
            pl.BlockSpec((1, 1), lambda b, f: (0, 0)),
        ],
        out_specs=pl.BlockSpec((1, _J, _TN, _F_BLK), lambda b, f: (b, 0, 0, f)),
        out_shape=jax.ShapeDtypeStruct((B, _J, _TN, _F), jnp.float32),
    )(frame, flx, fly, tcol,
      s11(tti_sigma), s11(a_max), s11(s_max), s11(reax_t))

    return out.transpose(0, 3, 2, 1)
